# baseline (speedup 1.0000x reference)
import jax
import jax.numpy as jnp
from jax import lax
from jax.experimental import pallas as pl
from jax.experimental.pallas import tpu as pltpu

N_TILE = 2048


def _mm_body(x_ref, m_ref, o_ref):
    o_ref[...] = lax.dot_general(
        x_ref[...], m_ref[...],
        dimension_numbers=(((1,), (1,)), ((), ())),
        preferred_element_type=jnp.float32)


def kernel(inputs, targets, mem):
    del targets
    b, f = inputs.shape
    n = mem.shape[0]
    return pl.pallas_call(
        _mm_body,
        grid=(pl.cdiv(n, N_TILE),),
        in_specs=[
            pl.BlockSpec((b, f), lambda i: (0, 0)),
            pl.BlockSpec((N_TILE, f), lambda i: (i, 0)),
        ],
        out_specs=pl.BlockSpec((b, N_TILE), lambda i: (0, i)),
        out_shape=jax.ShapeDtypeStruct((b, n), jnp.float32),
        compiler_params=pltpu.CompilerParams(
            dimension_semantics=("parallel",)),
    )(inputs, mem)
